# Initial kernel scaffold; baseline (speedup 1.0000x reference)
import jax
import jax.numpy as jnp
from jax.experimental import pallas as pl


def kernel(x, params, edge_index):
    raise NotImplementedError("write your pallas kernel here")


# speedup vs baseline: 1.8897x; 1.8897x over previous
import functools

import jax
import jax.numpy as jnp
from jax.experimental import pallas as pl

_N = 4096


def _ab_body(h_ref, wt_ref, wc_ref, bs_ref, a_ref, b_ref):
    h = h_ref[...]
    a_ref[...] = jnp.dot(h, wt_ref[...], preferred_element_type=jnp.float32)
    b_ref[...] = (
        jnp.dot(h, wc_ref[...], preferred_element_type=jnp.float32) + bs_ref[...]
    )


def _ab(h, wt, wc, bs):
    cin, cout = wt.shape
    rb = 512
    grid = (_N // rb,)
    return pl.pallas_call(
        _ab_body,
        grid=grid,
        in_specs=[
            pl.BlockSpec((rb, cin), lambda i: (i, 0)),
            pl.BlockSpec((cin, cout), lambda i: (0, 0)),
            pl.BlockSpec((cin, cout), lambda i: (0, 0)),
            pl.BlockSpec((1, cout), lambda i: (0, 0)),
        ],
        out_specs=[
            pl.BlockSpec((rb, cout), lambda i: (i, 0)),
            pl.BlockSpec((rb, cout), lambda i: (i, 0)),
        ],
        out_shape=[
            jax.ShapeDtypeStruct((_N, cout), jnp.float32),
            jax.ShapeDtypeStruct((_N, cout), jnp.float32),
        ],
    )(h, wt, wc, bs)


def _gram_body(e_ref, w_ref, o_ref):
    o_ref[...] = jnp.dot(
        e_ref[...], w_ref[...], preferred_element_type=jnp.float32
    )


def _gram(ecat, wint):
    rb, cb = 512, 1536
    out = pl.pallas_call(
        _gram_body,
        grid=(_N // rb, (3 * _N) // cb),
        in_specs=[
            pl.BlockSpec((rb, 24), lambda i, j: (i, 0)),
            pl.BlockSpec((24, cb), lambda i, j: (0, j)),
        ],
        out_specs=pl.BlockSpec((rb, cb), lambda i, j: (i, j)),
        out_shape=jax.ShapeDtypeStruct((_N, 3 * _N), jnp.float32),
    )(ecat, wint)
    return out.reshape(_N, _N, 3)


def _layer(h, p, src, dst):
    wt = p["Wt"]
    wc = p["Wp"] - wt
    bs = (p["bt"] + p["bp"]).reshape(1, -1)
    a, b = _ab(h, wt, wc, bs)
    agg = jax.ops.segment_max(a[src], dst, num_segments=_N)
    return jnp.where(jnp.isneginf(agg), 0.0, agg + b)


def kernel(x, params, edge_index):
    src, dst = edge_index[0], edge_index[1]
    h = x
    for p in params["shared"]:
        h = _layer(h, p, src, dst)
    shared = h
    outs = {}
    for name in ("node", "e1", "e2", "e3"):
        hh = shared
        for p in params[name]:
            hh = _layer(hh, p, src, dst)
        outs[name] = hh
    ecat = jnp.concatenate([outs["e1"], outs["e2"], outs["e3"]], axis=1)
    wint = jnp.zeros((3, _N, 3, 8), jnp.float32)
    wint = wint.at[0, :, 0, :].set(outs["e1"])
    wint = wint.at[1, :, 1, :].set(outs["e2"])
    wint = wint.at[2, :, 2, :].set(outs["e3"])
    wint = wint.transpose(0, 3, 1, 2).reshape(24, 3 * _N)
    m = _gram(ecat, wint)
    return (outs["node"], m)


# speedup vs baseline: 8.6000x; 4.5511x over previous
import functools

import jax
import jax.numpy as jnp
from jax import lax
from jax.experimental import pallas as pl
from jax.experimental.pallas import tpu as pltpu
from jax.experimental.pallas import tpu_sc as plsc

_N = 4096
_E = 262144
_C = 4096
_NEG = float("-inf")

_DN = lax.GatherDimensionNumbers(
    offset_dims=(), collapsed_slice_dims=(0,), start_index_map=(0,)
)


def _pad16(n):
    return (n + 15) // 16 * 16


def _bcast(v, e):
    return lax.gather(
        v,
        jnp.full((16, 1), e, jnp.int32),
        _DN,
        (1,),
        mode=lax.GatherScatterMode.PROMISE_IN_BOUNDS,
    )


def _scalar32(va, vb, w):
    val = jnp.int32(0)
    for k in range(16):
        val = jnp.where(w == k, va[k], val)
        val = jnp.where(w == k + 16, vb[k], val)
    return val


@functools.lru_cache(None)
def _segmax_sc(coutp):
    S = coutp // 16
    P = 32 // S
    R = _N // P
    mesh = plsc.VectorSubcoreMesh(core_axis_name="c", subcore_axis_name="s")

    def body(a_hbm, edges_hbm, meta_hbm, out_hbm, a_v, acc_v, ebuf_v, meta_v):
        c = lax.axis_index("c")
        s = lax.axis_index("s")
        w = s * 2 + c
        part = w // S
        sl = w % S
        row_lo = pl.multiple_of(part * R, R)
        pltpu.sync_copy(meta_hbm, meta_v)
        pltpu.sync_copy(a_hbm.at[sl], a_v)
        sa = _scalar32(meta_v[0:16], meta_v[16:32], w)
        nch = _scalar32(meta_v[32:48], meta_v[48:64], w)

        def ini(r, carry):
            acc_v[pl.ds(r * 16, 16)] = jnp.full((16,), _NEG, jnp.float32)
            return carry

        lax.fori_loop(0, R, ini, 0)
        iota = lax.broadcasted_iota(jnp.int32, (16,), 0)

        def chunk(g, carry):
            m0, d0 = carry
            off = pl.multiple_of(sa + g * _C, 16)
            pltpu.sync_copy(edges_hbm.at[pl.ds(off, _C)], ebuf_v)

            def group(q, carry):
                m, dprev = carry
                ev = ebuf_v[pl.ds(q * 16, 16)]
                for e in range(16):
                    wv = _bcast(ev, e)
                    dv = wv >> 12
                    sv = wv & 4095
                    a = plsc.load_gather(a_v, [(sv << 4) + iota])
                    mm = jnp.maximum(a, jnp.where(dv == dprev, m, _NEG))
                    ridx = dv - row_lo
                    ok = (ridx >= 0) & (ridx < R)
                    cidx = (jnp.clip(ridx, 0, R - 1) << 4) + iota
                    plsc.store_scatter(acc_v, [cidx], mm, mask=ok)
                    m, dprev = mm, dv
                return m, dprev

            return lax.fori_loop(0, _C // 16, group, (m0, d0))

        lax.fori_loop(
            0,
            nch,
            chunk,
            (jnp.full((16,), _NEG, jnp.float32), jnp.full((16,), -1, jnp.int32)),
        )
        pltpu.sync_copy(
            acc_v, out_hbm.at[sl, pl.ds(pl.multiple_of(row_lo * 16, 2048), R * 16)]
        )

    return pl.kernel(
        body,
        out_type=jax.ShapeDtypeStruct((S, _N * 16), jnp.float32),
        mesh=mesh,
        compiler_params=pltpu.CompilerParams(needs_layout_passes=False),
        scratch_types=[
            pltpu.VMEM((_N * 16,), jnp.float32),
            pltpu.VMEM((R * 16,), jnp.float32),
            pltpu.VMEM((_C,), jnp.int32),
            pltpu.VMEM((64,), jnp.int32),
        ],
    )


def _meta_for(sorted_packed, S):
    P = 32 // S
    R = _N // P
    keys = (jnp.arange(P + 1, dtype=jnp.int32) * R) << 12
    bnd = jnp.searchsorted(sorted_packed, keys, side="left").astype(jnp.int32)
    w = jnp.arange(32, dtype=jnp.int32)
    part = w // S
    start = bnd[part]
    end = bnd[part + 1]
    sa = start & ~15
    nch = (end - sa + _C - 1) // _C
    return jnp.concatenate([sa, nch])


def _ab_body(g_ref, bp_ref, wt_ref, wc_ref, bs_ref, a_ref, b_ref):
    g = g_ref[...]
    h = jnp.where(jnp.isneginf(g), 0.0, g + bp_ref[...])
    a_ref[...] = jnp.dot(
        h,
        wt_ref[...],
        preferred_element_type=jnp.float32,
        precision=lax.Precision.HIGHEST,
    )
    b_ref[...] = (
        jnp.dot(
            h,
            wc_ref[...],
            preferred_element_type=jnp.float32,
            precision=lax.Precision.HIGHEST,
        )
        + bs_ref[...]
    )


def _ab(agg, b, wt, wc, bs):
    cinp, coutp = wt.shape
    rb = 512
    return pl.pallas_call(
        _ab_body,
        grid=(_N // rb,),
        in_specs=[
            pl.BlockSpec((rb, cinp), lambda i: (i, 0)),
            pl.BlockSpec((rb, cinp), lambda i: (i, 0)),
            pl.BlockSpec((cinp, coutp), lambda i: (0, 0)),
            pl.BlockSpec((cinp, coutp), lambda i: (0, 0)),
            pl.BlockSpec((1, coutp), lambda i: (0, 0)),
        ],
        out_specs=[
            pl.BlockSpec((rb, coutp), lambda i: (i, 0)),
            pl.BlockSpec((rb, coutp), lambda i: (i, 0)),
        ],
        out_shape=[
            jax.ShapeDtypeStruct((_N, coutp), jnp.float32),
            jax.ShapeDtypeStruct((_N, coutp), jnp.float32),
        ],
    )(agg, b, wt, wc, bs)


def _comb_body(g_ref, bp_ref, h_ref):
    g = g_ref[...]
    h_ref[...] = jnp.where(jnp.isneginf(g), 0.0, g + bp_ref[...])


def _comb(agg, b):
    n, cp = agg.shape
    return pl.pallas_call(
        _comb_body,
        grid=(4,),
        in_specs=[
            pl.BlockSpec((n // 4, cp), lambda i: (i, 0)),
            pl.BlockSpec((n // 4, cp), lambda i: (i, 0)),
        ],
        out_specs=pl.BlockSpec((n // 4, cp), lambda i: (i, 0)),
        out_shape=jax.ShapeDtypeStruct((n, cp), jnp.float32),
    )(agg, b)


def _gram_body(e_ref, w_ref, o_ref):
    o_ref[...] = jnp.dot(
        e_ref[...],
        w_ref[...],
        preferred_element_type=jnp.float32,
        precision=lax.Precision.HIGHEST,
    )


def _gram(ecat, wint):
    rb, cb = 512, 1536
    out = pl.pallas_call(
        _gram_body,
        grid=(_N // rb, (3 * _N) // cb),
        in_specs=[
            pl.BlockSpec((rb, 24), lambda i, j: (i, 0)),
            pl.BlockSpec((24, cb), lambda i, j: (0, j)),
        ],
        out_specs=pl.BlockSpec((rb, cb), lambda i, j: (i, j)),
        out_shape=jax.ShapeDtypeStruct((_N, 3 * _N), jnp.float32),
    )(ecat, wint)
    return out.reshape(_N, _N, 3)


def _pad_params(p):
    cin, cout = p["Wt"].shape
    cinp, coutp = _pad16(cin), _pad16(cout)
    wt = jnp.zeros((cinp, coutp), jnp.float32).at[:cin, :cout].set(p["Wt"])
    wc = (
        jnp.zeros((cinp, coutp), jnp.float32)
        .at[:cin, :cout]
        .set(p["Wp"] - p["Wt"])
    )
    bs = (
        jnp.zeros((1, coutp), jnp.float32)
        .at[0, :cout]
        .set(p["bt"] + p["bp"])
    )
    return wt, wc, bs


def kernel(x, params, edge_index):
    src = edge_index[0]
    dst = edge_index[1]
    packed = (dst << 12) | src
    sorted_packed = jnp.sort(packed)
    edges = jnp.concatenate([sorted_packed, jnp.full((_C,), -1, jnp.int32)])
    metas = {s: _meta_for(sorted_packed, s) for s in (1, 2, 4, 8)}

    def step(state, p):
        agg, b = state
        wt, wc, bs = _pad_params(p)
        a, b2 = _ab(agg, b, wt, wc, bs)
        coutp = wt.shape[1]
        s_cnt = coutp // 16
        a3 = a.reshape(_N, s_cnt, 16).transpose(1, 0, 2).reshape(s_cnt, _N * 16)
        agg3 = _segmax_sc(coutp)(a3, edges, metas[s_cnt])
        agg2 = agg3.reshape(s_cnt, _N, 16).transpose(1, 0, 2).reshape(_N, coutp)
        return agg2, b2

    state = (x, jnp.zeros((_N, 128), jnp.float32))
    for p in params["shared"]:
        state = step(state, p)
    finals = {}
    for name in ("node", "e1", "e2", "e3"):
        st = state
        for p in params[name]:
            st = step(st, p)
        finals[name] = _comb(st[0], st[1])
    n_out = finals["node"][:, :7]
    e1, e2, e3 = (finals[k][:, :8] for k in ("e1", "e2", "e3"))
    ecat = jnp.concatenate([e1, e2, e3], axis=1)
    wint = jnp.zeros((3, _N, 3, 8), jnp.float32)
    wint = wint.at[0, :, 0, :].set(e1)
    wint = wint.at[1, :, 1, :].set(e2)
    wint = wint.at[2, :, 2, :].set(e3)
    wint = wint.transpose(0, 3, 1, 2).reshape(24, 3 * _N)
    m = _gram(ecat, wint)
    return (n_out, m)
